# baseline (speedup 1.0000x reference)
import jax
import jax.numpy as jnp
from jax.experimental import pallas as pl


def _lin_kernel(x_ref, xv_ref, wt_ref, wb_ref, b_ref, o_ref):
    acc = jnp.dot(x_ref[...], wt_ref[...], preferred_element_type=jnp.float32,
                  precision=jax.lax.Precision.HIGHEST)
    acc += jnp.dot(xv_ref[...], wb_ref[...], preferred_element_type=jnp.float32,
                   precision=jax.lax.Precision.HIGHEST)
    o_ref[...] = acc + b_ref[...]


def kernel(x, edge_index, edge_weight, W_v2e, W_e2v, W_lin, b_lin):
    N_V = x.shape[0]
    N_E = N_V
    src = edge_index[0]
    dst = edge_index[1]
    h1 = x @ W_v2e
    m1 = edge_weight[:, None] * jnp.take(h1, src, axis=0)
    x_e = jax.ops.segment_sum(m1, dst, num_segments=N_E)
    h2 = x_e @ W_e2v
    m2 = edge_weight[:, None] * jnp.take(h2, dst, axis=0)
    x_v = jax.ops.segment_sum(m2, src, num_segments=N_V)

    B = 1000
    grid = (N_V // B,)
    out = pl.pallas_call(
        _lin_kernel,
        grid=grid,
        in_specs=[
            pl.BlockSpec((B, 128), lambda i: (i, 0)),
            pl.BlockSpec((B, 128), lambda i: (i, 0)),
            pl.BlockSpec((128, 128), lambda i: (0, 0)),
            pl.BlockSpec((128, 128), lambda i: (0, 0)),
            pl.BlockSpec((1, 128), lambda i: (0, 0)),
        ],
        out_specs=pl.BlockSpec((B, 128), lambda i: (i, 0)),
        out_shape=jax.ShapeDtypeStruct((N_V, 128), jnp.float32),
    )(x, x_v, W_lin[:128], W_lin[128:], b_lin[None, :])
    return out


# speedup vs baseline: 6.9569x; 6.9569x over previous
import functools

import jax
import jax.numpy as jnp
from jax import lax
from jax.experimental import pallas as pl
from jax.experimental.pallas import tpu as pltpu
from jax.experimental.pallas import tpu_sc as plsc

N = 10000
D = 128
E = 320000
NC = 2
NS = 16
C = 128
K = 80
E_PAD = NC * NS * C * K
N_PAD = 10240
ROWS_PER_TILE = N_PAD // NS


def _sc_phase_body(tbl_hbm, gidx_hbm, sidx_hbm, w_hbm, out_hbm,
                   gidx_v, sidx_v, w_v, rows_v, acc, sem):
    c = lax.axis_index("c")
    s = lax.axis_index("s")
    wid = c * NS + s
    row0 = pl.multiple_of(s * ROWS_PER_TILE, C)
    idx0 = pl.multiple_of(wid * K, 8)

    def zero_body(i, carry):
        for q in range(D // 16):
            rows_v[i, pl.ds(q * 16, 16)] = jnp.zeros((16,), jnp.float32)
        return carry

    lax.fori_loop(0, C, zero_body, 0)
    for p in range(ROWS_PER_TILE // C):
        pltpu.sync_copy(rows_v, acc.at[pl.ds(row0 + p * C, C)])
    plsc.subcore_barrier()

    pltpu.sync_copy(gidx_hbm.at[pl.ds(idx0, K)], gidx_v)
    pltpu.sync_copy(sidx_hbm.at[pl.ds(idx0, K)], sidx_v)
    pltpu.sync_copy(w_hbm.at[pl.ds(idx0, K)], w_v)

    def chunk_body(j, carry):
        pltpu.async_copy(tbl_hbm.at[gidx_v.at[j]], rows_v, sem).wait()

        def grp_body(g, carry2):
            w16 = w_v[j, pl.ds(g * 16, 16)]
            for e in range(16):
                wsp = w16.at[jnp.full((16,), e, jnp.int32)].get(
                    mode="promise_in_bounds")
                r = g * 16 + e
                for q in range(D // 16):
                    v = rows_v[r, pl.ds(q * 16, 16)]
                    rows_v[r, pl.ds(q * 16, 16)] = v * wsp
            return carry2

        lax.fori_loop(0, C // 16, grp_body, 0)

        pltpu.sync_copy(rows_v, acc.at[sidx_v.at[j]], add=True)
        return carry

    lax.fori_loop(0, K, chunk_body, 0)
    plsc.subcore_barrier()

    pltpu.sync_copy(acc.at[pl.ds(row0, ROWS_PER_TILE)],
                    out_hbm.at[c].at[pl.ds(row0, ROWS_PER_TILE)])


_sc_phase = functools.partial(
    pl.kernel,
    mesh=plsc.VectorSubcoreMesh(core_axis_name="c", subcore_axis_name="s"),
    out_type=jax.ShapeDtypeStruct((NC, N_PAD, D), jnp.float32),
    scratch_types=[
        pltpu.VMEM((K, C), jnp.int32),
        pltpu.VMEM((K, C), jnp.int32),
        pltpu.VMEM((K, C), jnp.float32),
        pltpu.VMEM((C, D), jnp.float32),
        pltpu.VMEM_SHARED((N_PAD, D), jnp.float32),
        pltpu.SemaphoreType.DMA,
    ],
)(_sc_phase_body)


def _mm_kernel(a_ref, w_ref, o_ref):
    o_ref[...] = jnp.dot(a_ref[...], w_ref[...],
                         preferred_element_type=jnp.float32,
                         precision=lax.Precision.HIGHEST)


def _mm2_kernel(p_ref, w_ref, o_ref):
    a = p_ref[0] + p_ref[1]
    o_ref[...] = jnp.dot(a, w_ref[...], preferred_element_type=jnp.float32,
                         precision=lax.Precision.HIGHEST)


def _lin_kernel(x_ref, p_ref, wt_ref, wb_ref, b_ref, o_ref):
    acc = jnp.dot(x_ref[...], wt_ref[...], preferred_element_type=jnp.float32,
                  precision=lax.Precision.HIGHEST)
    acc += jnp.dot(p_ref[0] + p_ref[1], wb_ref[...],
                   preferred_element_type=jnp.float32,
                   precision=lax.Precision.HIGHEST)
    o_ref[...] = acc + b_ref[...]


_B = 1000
_G = N // _B


def _mm(x, w):
    return pl.pallas_call(
        _mm_kernel,
        grid=(_G,),
        in_specs=[pl.BlockSpec((_B, D), lambda i: (i, 0)),
                  pl.BlockSpec((D, D), lambda i: (0, 0))],
        out_specs=pl.BlockSpec((_B, D), lambda i: (i, 0)),
        out_shape=jax.ShapeDtypeStruct((N, D), jnp.float32),
    )(x, w)


def _mm2(parts, w):
    return pl.pallas_call(
        _mm2_kernel,
        grid=(_G,),
        in_specs=[pl.BlockSpec((NC, _B, D), lambda i: (0, i, 0)),
                  pl.BlockSpec((D, D), lambda i: (0, 0))],
        out_specs=pl.BlockSpec((_B, D), lambda i: (i, 0)),
        out_shape=jax.ShapeDtypeStruct((N, D), jnp.float32),
    )(parts, w)


def _lin(x, parts, wt, wb, b):
    return pl.pallas_call(
        _lin_kernel,
        grid=(_G,),
        in_specs=[pl.BlockSpec((_B, D), lambda i: (i, 0)),
                  pl.BlockSpec((NC, _B, D), lambda i: (0, i, 0)),
                  pl.BlockSpec((D, D), lambda i: (0, 0)),
                  pl.BlockSpec((D, D), lambda i: (0, 0)),
                  pl.BlockSpec((1, D), lambda i: (0, 0))],
        out_specs=pl.BlockSpec((_B, D), lambda i: (i, 0)),
        out_shape=jax.ShapeDtypeStruct((N, D), jnp.float32),
    )(x, parts, wt, wb, b)


def kernel(x, edge_index, edge_weight, W_v2e, W_e2v, W_lin, b_lin):
    src = edge_index[0]
    dst = edge_index[1]
    pad = E_PAD - E
    pad_idx = (jnp.arange(pad, dtype=jnp.int32) * 97) % N
    srcp = jnp.concatenate([src, pad_idx]).reshape(E_PAD // C, C)
    dstp = jnp.concatenate([dst, pad_idx]).reshape(E_PAD // C, C)
    wp = jnp.concatenate(
        [edge_weight, jnp.zeros((pad,), jnp.float32)]).reshape(E_PAD // C, C)

    h1 = _mm(x, W_v2e)
    xe_parts = _sc_phase(h1, srcp, dstp, wp)
    h2 = _mm2(xe_parts, W_e2v)
    xv_parts = _sc_phase(h2, dstp, srcp, wp)
    return _lin(x, xv_parts, W_lin[:D], W_lin[D:], b_lin[None, :])
